# baseline (speedup 1.0000x reference)
import functools

import jax
import jax.numpy as jnp
from jax import lax
from jax.experimental import pallas as pl
from jax.experimental.pallas import tpu as pltpu
from jax.experimental.pallas import tpu_sc as plsc

D_MODEL = 1024
SCALE = 32.0
NC, NS, L = 2, 16, 16
NW = NC * NS
B = 4 * 8192
BPW = B // NW
CH = 16
NCHUNK = BPW // CH
RING = 3
VPR = D_MODEL // L

_mesh = plsc.VectorSubcoreMesh(core_axis_name="c", subcore_axis_name="s")


@functools.partial(
    pl.kernel,
    out_type=jax.ShapeDtypeStruct((B, D_MODEL), jnp.float32),
    mesh=_mesh,
    scratch_types=[
        pltpu.VMEM((BPW,), jnp.int32),
    ] + [pltpu.VMEM((CH, D_MODEL), jnp.float32)] * RING
      + [pltpu.VMEM_SHARED((NS, RING, CH, D_MODEL), jnp.float32)]
      + [pltpu.SemaphoreType.DMA] * (3 * RING),
)
def _embed_sc(x_hbm, table_hbm, out_hbm, idx_v, *rest):
    bufs = rest[:RING]
    shared = rest[RING]
    gsems = rest[RING + 1:2 * RING + 1]
    xsems = rest[2 * RING + 1:3 * RING + 1]
    wsems = rest[3 * RING + 1:]

    sid = lax.axis_index("s")
    wid = sid * NC + lax.axis_index("c")
    base = wid * BPW
    pltpu.sync_copy(x_hbm.at[pl.ds(base, BPW)], idx_v)

    def issue_gather(c, b):
        off = pl.multiple_of(c * CH, 8)
        pltpu.async_copy(table_hbm.at[idx_v.at[pl.ds(off, CH)]], bufs[b], gsems[b])

    def wait_gather(b):
        pltpu.make_async_copy(table_hbm.at[pl.ds(0, CH)], bufs[b], gsems[b]).wait()

    def scale_buf(b):
        buf = bufs[b]

        @plsc.parallel_loop(0, CH)
        def _(r):
            for j in range(VPR):
                buf[r, pl.ds(j * L, L)] = buf[r, pl.ds(j * L, L)] * SCALE

    def issue_xbar(b):
        pltpu.async_copy(bufs[b], shared.at[sid, b], xsems[b])

    def wait_xbar(b):
        pltpu.make_async_copy(bufs[b], shared.at[sid, b], xsems[b]).wait()

    def issue_write(c, b):
        off = pl.multiple_of(c * CH, 8)
        pltpu.async_copy(shared.at[sid, b], out_hbm.at[pl.ds(base + off, CH)], wsems[b])

    def wait_write(b):
        pltpu.make_async_copy(
            shared.at[sid, b], out_hbm.at[pl.ds(0, CH)], wsems[b]
        ).wait()

    issue_gather(0, 0)
    issue_gather(1, 1)

    wait_gather(0)
    scale_buf(0)
    issue_xbar(0)
    issue_gather(2, 2)

    wait_gather(1)
    scale_buf(1)
    issue_xbar(1)
    wait_xbar(0)
    issue_write(0, 0)
    issue_gather(3, 0)

    for c in (2, 3):
        b = c % RING
        pb = (b + 2) % RING
        wait_gather(b)
        scale_buf(b)
        if c >= RING:
            wait_write(b)
        issue_xbar(b)
        wait_xbar(pb)
        issue_write(c - 1, pb)
        issue_gather(c + 2, pb)

    def outer(t, carry):
        for i in range(RING):
            c = 4 + t * RING + i
            b = (4 + i) % RING
            pb = (b + 2) % RING

            wait_gather(b)
            scale_buf(b)

            wait_write(b)

            issue_xbar(b)
            wait_xbar(pb)
            issue_write(c - 1, pb)

            @pl.when(c + 2 < NCHUNK)
            def _():
                issue_gather(c + 2, pb)

        return carry

    lax.fori_loop(0, (NCHUNK - 4) // RING, outer, 0)

    last = NCHUNK - 1
    lb = last % RING
    wait_xbar(lb)
    issue_write(last, lb)
    for b in range(RING):
        wait_write(b)


def kernel(x, embedding):
    xf = x.reshape(-1).astype(jnp.int32)
    out = _embed_sc(xf, embedding)
    return out.reshape(x.shape[0], x.shape[1], D_MODEL)
